# Initial kernel scaffold; baseline (speedup 1.0000x reference)
import jax
import jax.numpy as jnp
from jax.experimental import pallas as pl


def kernel(x, Wg, bg, W1, b1, W2, b2):
    raise NotImplementedError("write your pallas kernel here")


# speedup vs baseline: 1.3399x; 1.3399x over previous
import jax
import jax.numpy as jnp
from jax.experimental import pallas as pl
from jax.experimental.pallas import tpu as pltpu

_B, _D, _F, _E = 2048, 768, 2048, 8
_LANES = 128


def _gate_body(x_ref, wg_ref, bg_ref, g_ref):
    logits = jnp.dot(x_ref[...], wg_ref[...],
                     preferred_element_type=jnp.float32) + bg_ref[...]
    m = jnp.max(logits, axis=-1, keepdims=True)
    ex = jnp.exp(logits - m)
    w = ex / jnp.sum(ex, axis=-1, keepdims=True)
    lane = jax.lax.broadcasted_iota(jnp.int32, w.shape, 1)
    m1 = jnp.max(w, axis=-1, keepdims=True)
    i1 = jnp.min(jnp.where(w >= m1, lane, jnp.int32(1 << 30)), axis=-1,
                 keepdims=True)
    w_masked = jnp.where(lane == i1, -1.0, w)
    m2 = jnp.max(w_masked, axis=-1, keepdims=True)
    i2 = jnp.min(jnp.where(w_masked >= m2, lane, jnp.int32(1 << 30)), axis=-1,
                 keepdims=True)
    g_ref[...] = jnp.where((lane == i1) | (lane == i2), w, 0.0)


def _moe_body(xb_ref, g_ref, w1_ref, b1_ref, w2_ref, b2_ref, o_ref):
    e = pl.program_id(1)
    h = jnp.dot(xb_ref[...], w1_ref[0], preferred_element_type=jnp.float32)
    h = jnp.maximum(h + b1_ref[0], 0.0).astype(jnp.bfloat16)
    y = jnp.dot(h, w2_ref[0], preferred_element_type=jnp.float32) + b2_ref[0]
    lane = jax.lax.broadcasted_iota(jnp.int32, g_ref.shape, 1)
    ge = jnp.sum(jnp.where(lane == e, g_ref[...], 0.0), axis=1, keepdims=True)
    contrib = ge * y

    @pl.when(e == 0)
    def _():
        o_ref[...] = contrib

    @pl.when(e != 0)
    def _():
        o_ref[...] += contrib


def kernel(x, Wg, bg, W1, b1, W2, b2):
    wg_pad = jnp.zeros((_D, _LANES), jnp.float32).at[:, :_E].set(Wg)
    bg_pad = jnp.full((1, _LANES), -1e30, jnp.float32).at[0, :_E].set(bg)

    gates = pl.pallas_call(
        _gate_body,
        out_shape=jax.ShapeDtypeStruct((_B, _LANES), jnp.float32),
    )(x, wg_pad, bg_pad)

    xb = x.astype(jnp.bfloat16)
    w1b = W1.astype(jnp.bfloat16)
    w2b = W2.astype(jnp.bfloat16)

    bt = _B // 2
    out = pl.pallas_call(
        _moe_body,
        grid=(_B // bt, _E),
        in_specs=[
            pl.BlockSpec((bt, _D), lambda t, e: (t, 0)),
            pl.BlockSpec((bt, _LANES), lambda t, e: (t, 0)),
            pl.BlockSpec((1, _D, _F), lambda t, e: (e, 0, 0)),
            pl.BlockSpec((1, 1, _F), lambda t, e: (e, 0, 0)),
            pl.BlockSpec((1, _F, _D), lambda t, e: (e, 0, 0)),
            pl.BlockSpec((1, 1, _D), lambda t, e: (e, 0, 0)),
        ],
        out_specs=pl.BlockSpec((bt, _D), lambda t, e: (t, 0)),
        out_shape=jax.ShapeDtypeStruct((_B, _D), jnp.float32),
        compiler_params=pltpu.CompilerParams(
            dimension_semantics=("arbitrary", "arbitrary")),
    )(xb, gates, w1b, b1.reshape(_E, 1, _F), w2b, b2.reshape(_E, 1, _D))
    return out


# speedup vs baseline: 1.3822x; 1.0316x over previous
import functools

import jax
import jax.numpy as jnp
from jax import lax
from jax.experimental import pallas as pl
from jax.experimental.pallas import tpu as pltpu
from jax.experimental.pallas import tpu_sc as plsc

_B, _D, _F, _E = 2048, 768, 2048, 8
_S = 2 * _B
_T = 256
_NT = _S // _T
_G = _NT + _E - 1
_LANES = 128
_SROWS = _S // _LANES


def _gate_body(x_ref, wg_ref, bg_ref, e_ref, w_ref):
    logits = jnp.dot(x_ref[...], wg_ref[...],
                     preferred_element_type=jnp.float32) + bg_ref[...]
    m = jnp.max(logits, axis=-1, keepdims=True)
    ex = jnp.exp(logits - m)
    w = ex / jnp.sum(ex, axis=-1, keepdims=True)
    lane = lax.broadcasted_iota(jnp.int32, w.shape, 1)
    m1 = jnp.max(w, axis=-1, keepdims=True)
    i1 = jnp.min(jnp.where(w >= m1, lane, jnp.int32(1 << 30)), axis=-1,
                 keepdims=True)
    wm = jnp.where(lane == i1, -1.0, w)
    m2 = jnp.max(wm, axis=-1, keepdims=True)
    i2 = jnp.min(jnp.where(wm >= m2, lane, jnp.int32(1 << 30)), axis=-1,
                 keepdims=True)
    lane8 = lax.broadcasted_iota(jnp.int32, e_ref.shape, 1)
    e_ref[...] = jnp.where(lane8 == 0, i1, 0) + jnp.where(lane8 == 1, i2, 0)
    w_ref[...] = (jnp.where(lane8 == 0, m1, 0.0)
                  + jnp.where(lane8 == 1, m2, 0.0))


def _sort_body(es_ref, pos_ref, meta_ref):
    es = es_ref[...]
    rr = lax.broadcasted_iota(jnp.int32, (_SROWS, _SROWS), 0)
    rc = lax.broadcasted_iota(jnp.int32, (_SROWS, _SROWS), 1)
    tl = jnp.where(rc < rr, 1.0, 0.0)
    ur = lax.broadcasted_iota(jnp.int32, (_LANES, _LANES), 0)
    uc = lax.broadcasted_iota(jnp.int32, (_LANES, _LANES), 1)
    uu = jnp.where(ur < uc, 1.0, 0.0)
    lane = lax.broadcasted_iota(jnp.int32, es.shape, 1)
    c_mat = jnp.zeros((_SROWS, _LANES), jnp.float32)
    for e in range(_E):
        mask = (es == e).astype(jnp.float32)
        c_mat = c_mat + jnp.sum(mask, axis=1, keepdims=True) * (
            jnp.where(lane == e, 1.0, 0.0))
    rowcum = jnp.dot(tl, c_mat, preferred_element_type=jnp.float32)
    tot = jnp.sum(c_mat, axis=0, keepdims=True)
    lane1 = lax.broadcasted_iota(jnp.int32, tot.shape, 1)
    offs = jnp.zeros(tot.shape, jnp.float32)
    run = jnp.zeros((1, 1), jnp.float32)
    for e in range(_E):
        offs = offs + jnp.where(lane1 == e, run, 0.0)
        run = run + jnp.sum(jnp.where(lane1 == e, tot, 0.0), axis=1,
                            keepdims=True)
    pos_f = jnp.zeros(es.shape, jnp.float32)
    for e in range(_E):
        mask = (es == e).astype(jnp.float32)
        pref = jnp.dot(mask, uu, preferred_element_type=jnp.float32)
        ext = jnp.sum(jnp.where(lane == e, rowcum, 0.0), axis=1,
                      keepdims=True)
        off_e = jnp.sum(jnp.where(lane1 == e, offs, 0.0), axis=1,
                        keepdims=True)
        pos_f = pos_f + mask * (pref + ext + off_e)
    pos_ref[...] = pos_f.astype(jnp.int32)
    row8 = lax.broadcasted_iota(jnp.int32, meta_ref.shape, 0)
    meta_ref[...] = (jnp.where(row8 == 0, jnp.broadcast_to(offs, meta_ref.shape), 0.0)
                     + jnp.where(row8 == 1, jnp.broadcast_to(tot, meta_ref.shape), 0.0)
                     ).astype(jnp.int32)


def _ffn_body(t_ref, e_ref, lo_ref, hi_ref, xs_ref, w1_ref, b1_ref, w2_ref,
              b2_ref, ys_ref):
    g = pl.program_id(0)
    h = jnp.dot(xs_ref[...].astype(jnp.bfloat16), w1_ref[0],
                preferred_element_type=jnp.float32)
    h = jnp.maximum(h + b1_ref[0], 0.0).astype(jnp.bfloat16)
    y = jnp.dot(h, w2_ref[0], preferred_element_type=jnp.float32) + b2_ref[0]
    row = lax.broadcasted_iota(jnp.int32, (_T, 1), 0)
    y = jnp.where((row >= lo_ref[g]) & (row < hi_ref[g]), y, 0.0)
    first = jnp.logical_or(g == 0, t_ref[g] != t_ref[jnp.maximum(g - 1, 0)])

    @pl.when(first)
    def _():
        ys_ref[...] = y

    @pl.when(jnp.logical_not(first))
    def _():
        ys_ref[...] += y


def _comb_body(a_ref, b_ref, wa_ref, wb_ref, o_ref):
    o_ref[...] = wa_ref[...] * a_ref[...] + wb_ref[...] * b_ref[...]


def _sc_dispatch(x, pos_slot):
    mesh = plsc.VectorSubcoreMesh(core_axis_name="c", subcore_axis_name="s")

    @functools.partial(
        pl.kernel, mesh=mesh,
        out_type=jax.ShapeDtypeStruct((_S, _D), jnp.float32),
        scratch_types=[pltpu.VMEM((_LANES,), jnp.int32),
                       pltpu.VMEM((_LANES, _D), jnp.float32),
                       pltpu.SemaphoreType.DMA],
    )
    def k(x_hbm, pos_hbm, xs_hbm, idx_v, rows_v, sem):
        wid = lax.axis_index("s") * 2 + lax.axis_index("c")
        pltpu.sync_copy(pos_hbm.at[wid], idx_v)
        base = lax.rem(wid, 16) * _LANES
        pltpu.async_copy(x_hbm.at[pl.ds(base, _LANES)], rows_v, sem).wait()
        pltpu.sync_copy(rows_v, xs_hbm.at[idx_v])

    return k(x, pos_slot)


def _sc_combine(ys, pa, pb):
    mesh = plsc.VectorSubcoreMesh(core_axis_name="c", subcore_axis_name="s")
    otype = (jax.ShapeDtypeStruct((_B, _D), jnp.float32),
             jax.ShapeDtypeStruct((_B, _D), jnp.float32))

    @functools.partial(
        pl.kernel, mesh=mesh, out_type=otype,
        scratch_types=[pltpu.VMEM((64,), jnp.int32),
                       pltpu.VMEM((64,), jnp.int32),
                       pltpu.VMEM((64, _D), jnp.float32),
                       pltpu.VMEM((64, _D), jnp.float32),
                       pltpu.SemaphoreType.DMA,
                       pltpu.SemaphoreType.DMA],
    )
    def k(ys_hbm, pa_hbm, pb_hbm, a_hbm, b_hbm, ia_v, ib_v, ra_v, rb_v,
          sa, sb):
        wid = lax.axis_index("s") * 2 + lax.axis_index("c")
        pltpu.sync_copy(pa_hbm.at[wid], ia_v)
        pltpu.sync_copy(pb_hbm.at[wid], ib_v)
        ca = pltpu.async_copy(ys_hbm.at[ia_v], ra_v, sa)
        cb = pltpu.async_copy(ys_hbm.at[ib_v], rb_v, sb)
        ca.wait()
        pltpu.sync_copy(ra_v, a_hbm.at[pl.ds(wid * 64, 64)])
        cb.wait()
        pltpu.sync_copy(rb_v, b_hbm.at[pl.ds(wid * 64, 64)])

    return k(ys, pa, pb)


def kernel(x, Wg, bg, W1, b1, W2, b2):
    bt = _B // 8
    wg_pad = jnp.zeros((_D, _LANES), jnp.float32).at[:, :_E].set(Wg)
    bg_pad = jnp.full((1, _LANES), -1e30, jnp.float32).at[0, :_E].set(bg)

    eout, wout = pl.pallas_call(
        _gate_body,
        grid=(8,),
        in_specs=[
            pl.BlockSpec((bt, _D), lambda t: (t, 0)),
            pl.BlockSpec((_D, _LANES), lambda t: (0, 0)),
            pl.BlockSpec((1, _LANES), lambda t: (0, 0)),
        ],
        out_specs=[
            pl.BlockSpec((bt, _E), lambda t: (t, 0)),
            pl.BlockSpec((bt, _E), lambda t: (t, 0)),
        ],
        out_shape=(jax.ShapeDtypeStruct((_B, _E), jnp.int32),
                   jax.ShapeDtypeStruct((_B, _E), jnp.float32)),
    )(x, wg_pad, bg_pad)

    e_slot = jnp.concatenate([eout[:, 0], eout[:, 1]]).reshape(_SROWS, _LANES)

    pos_slot, meta = pl.pallas_call(
        _sort_body,
        out_shape=(jax.ShapeDtypeStruct((_SROWS, _LANES), jnp.int32),
                   jax.ShapeDtypeStruct((8, _LANES), jnp.int32)),
    )(e_slot)

    offs = meta[0, :_E]
    cnt = meta[1, :_E]
    end = offs + cnt
    t0 = offs // _T
    t1 = jnp.where(cnt > 0, (end - 1) // _T, t0 - 1)
    ni = jnp.maximum(t1 - t0 + 1, 0)
    base = jnp.concatenate([jnp.zeros((1,), ni.dtype), jnp.cumsum(ni)])
    gidx = jnp.arange(_G, dtype=jnp.int32)
    e_of = jnp.sum((gidx[:, None] >= base[None, 1:]).astype(jnp.int32),
                   axis=1)
    valid = gidx < base[_E]
    e_of = jnp.clip(e_of, 0, _E - 1)
    t_of = t0[e_of] + (gidx - base[e_of])
    t_of = jnp.where(valid, t_of, _NT - 1)
    rs = t_of * _T
    lo = jnp.where(valid, jnp.clip(offs[e_of] - rs, 0, _T), 0)
    hi = jnp.where(valid, jnp.clip(end[e_of] - rs, 0, _T), 0)
    wi_t = t_of.astype(jnp.int32)
    wi_e = jnp.where(valid, e_of, _E - 1).astype(jnp.int32)

    xs = _sc_dispatch(x, pos_slot)

    w1b = W1.astype(jnp.bfloat16)
    w2b = W2.astype(jnp.bfloat16)
    ys = pl.pallas_call(
        _ffn_body,
        grid_spec=pltpu.PrefetchScalarGridSpec(
            num_scalar_prefetch=4,
            grid=(_G,),
            in_specs=[
                pl.BlockSpec((_T, _D), lambda g, t, e, lo_, hi_: (t[g], 0)),
                pl.BlockSpec((1, _D, _F),
                             lambda g, t, e, lo_, hi_: (e[g], 0, 0)),
                pl.BlockSpec((1, 1, _F),
                             lambda g, t, e, lo_, hi_: (e[g], 0, 0)),
                pl.BlockSpec((1, _F, _D),
                             lambda g, t, e, lo_, hi_: (e[g], 0, 0)),
                pl.BlockSpec((1, 1, _D),
                             lambda g, t, e, lo_, hi_: (e[g], 0, 0)),
            ],
            out_specs=pl.BlockSpec((_T, _D),
                                   lambda g, t, e, lo_, hi_: (t[g], 0)),
        ),
        out_shape=jax.ShapeDtypeStruct((_S, _D), jnp.float32),
        compiler_params=pltpu.CompilerParams(
            dimension_semantics=("arbitrary",)),
    )(wi_t, wi_e, lo.astype(jnp.int32), hi.astype(jnp.int32),
      xs, w1b, b1.reshape(_E, 1, _F), w2b, b2.reshape(_E, 1, _D))

    pa = pos_slot[:_SROWS // 2].reshape(32, 64)
    pb = pos_slot[_SROWS // 2:].reshape(32, 64)
    a_rows, b_rows = _sc_combine(ys, pa, pb)

    out = pl.pallas_call(
        _comb_body,
        grid=(4,),
        in_specs=[
            pl.BlockSpec((_B // 4, _D), lambda t: (t, 0)),
            pl.BlockSpec((_B // 4, _D), lambda t: (t, 0)),
            pl.BlockSpec((_B // 4, 1), lambda t: (t, 0)),
            pl.BlockSpec((_B // 4, 1), lambda t: (t, 0)),
        ],
        out_specs=pl.BlockSpec((_B // 4, _D), lambda t: (t, 0)),
        out_shape=jax.ShapeDtypeStruct((_B, _D), jnp.float32),
    )(a_rows, b_rows, wout[:, 0:1], wout[:, 1:2])
    return out


# speedup vs baseline: 1.7560x; 1.2704x over previous
import functools

import jax
import jax.numpy as jnp
from jax import lax
from jax.experimental import pallas as pl
from jax.experimental.pallas import tpu as pltpu
from jax.experimental.pallas import tpu_sc as plsc

_B, _D, _F, _E = 2048, 768, 2048, 8
_S = 2 * _B
_T = 256
_NT = _S // _T
_G = _NT + _E - 1
_LANES = 128
_SROWS = _S // _LANES


def _gate_body(x_ref, wg_ref, bg_ref, e_ref, w_ref):
    logits = jnp.dot(x_ref[...], wg_ref[...],
                     preferred_element_type=jnp.float32) + bg_ref[...]
    m = jnp.max(logits, axis=-1, keepdims=True)
    ex = jnp.exp(logits - m)
    w = ex / jnp.sum(ex, axis=-1, keepdims=True)
    lane = lax.broadcasted_iota(jnp.int32, w.shape, 1)
    m1 = jnp.max(w, axis=-1, keepdims=True)
    i1 = jnp.min(jnp.where(w >= m1, lane, jnp.int32(1 << 30)), axis=-1,
                 keepdims=True)
    wm = jnp.where(lane == i1, -1.0, w)
    m2 = jnp.max(wm, axis=-1, keepdims=True)
    i2 = jnp.min(jnp.where(wm >= m2, lane, jnp.int32(1 << 30)), axis=-1,
                 keepdims=True)
    lane8 = lax.broadcasted_iota(jnp.int32, e_ref.shape, 1)
    e_ref[...] = jnp.where(lane8 == 0, i1, 0) + jnp.where(lane8 == 1, i2, 0)
    w_ref[...] = (jnp.where(lane8 == 0, m1, 0.0)
                  + jnp.where(lane8 == 1, m2, 0.0))


def _sort_body(es_ref, pos_ref, meta_ref):
    es = es_ref[...]
    rr = lax.broadcasted_iota(jnp.int32, (_SROWS, _SROWS), 0)
    rc = lax.broadcasted_iota(jnp.int32, (_SROWS, _SROWS), 1)
    tl = jnp.where(rc < rr, 1.0, 0.0)
    ur = lax.broadcasted_iota(jnp.int32, (_LANES, _LANES), 0)
    uc = lax.broadcasted_iota(jnp.int32, (_LANES, _LANES), 1)
    uu = jnp.where(ur < uc, 1.0, 0.0)
    lane = lax.broadcasted_iota(jnp.int32, es.shape, 1)
    c_mat = jnp.zeros((_SROWS, _LANES), jnp.float32)
    for e in range(_E):
        mask = (es == e).astype(jnp.float32)
        c_mat = c_mat + jnp.sum(mask, axis=1, keepdims=True) * (
            jnp.where(lane == e, 1.0, 0.0))
    rowcum = jnp.dot(tl, c_mat, preferred_element_type=jnp.float32)
    tot = jnp.sum(c_mat, axis=0, keepdims=True)
    lane1 = lax.broadcasted_iota(jnp.int32, tot.shape, 1)
    offs = jnp.zeros(tot.shape, jnp.float32)
    run = jnp.zeros((1, 1), jnp.float32)
    for e in range(_E):
        offs = offs + jnp.where(lane1 == e, run, 0.0)
        run = run + jnp.sum(jnp.where(lane1 == e, tot, 0.0), axis=1,
                            keepdims=True)
    pos_f = jnp.zeros(es.shape, jnp.float32)
    for e in range(_E):
        mask = (es == e).astype(jnp.float32)
        pref = jnp.dot(mask, uu, preferred_element_type=jnp.float32)
        ext = jnp.sum(jnp.where(lane == e, rowcum, 0.0), axis=1,
                      keepdims=True)
        off_e = jnp.sum(jnp.where(lane1 == e, offs, 0.0), axis=1,
                        keepdims=True)
        pos_f = pos_f + mask * (pref + ext + off_e)
    pos_ref[...] = pos_f.astype(jnp.int32)
    row8 = lax.broadcasted_iota(jnp.int32, meta_ref.shape, 0)
    meta_ref[...] = (jnp.where(row8 == 0, jnp.broadcast_to(offs, meta_ref.shape), 0.0)
                     + jnp.where(row8 == 1, jnp.broadcast_to(tot, meta_ref.shape), 0.0)
                     ).astype(jnp.int32)


def _ffn_body(t_ref, e_ref, lo_ref, hi_ref, xs_ref, w1_ref, b1_ref, w2_ref,
              b2_ref, ys_ref):
    g = pl.program_id(0)
    h = jnp.dot(xs_ref[...], w1_ref[0], preferred_element_type=jnp.float32)
    h = jnp.maximum(h + b1_ref[0], 0.0)
    y = jnp.dot(h, w2_ref[0], preferred_element_type=jnp.float32) + b2_ref[0]
    row = lax.broadcasted_iota(jnp.int32, (_T, 1), 0)
    y = jnp.where((row >= lo_ref[g]) & (row < hi_ref[g]), y, 0.0)
    first = jnp.logical_or(g == 0, t_ref[g] != t_ref[jnp.maximum(g - 1, 0)])

    @pl.when(first)
    def _():
        ys_ref[...] = y

    @pl.when(jnp.logical_not(first))
    def _():
        ys_ref[...] += y


def _comb_body(a_ref, b_ref, wa_ref, wb_ref, o_ref):
    o_ref[...] = wa_ref[...] * a_ref[...] + wb_ref[...] * b_ref[...]


def _sc_dispatch(x, pos_slot):
    mesh = plsc.VectorSubcoreMesh(core_axis_name="c", subcore_axis_name="s")

    @functools.partial(
        pl.kernel, mesh=mesh,
        out_type=jax.ShapeDtypeStruct((_S, _D), jnp.float32),
        scratch_types=[pltpu.VMEM((_LANES,), jnp.int32),
                       pltpu.VMEM((_LANES, _D), jnp.float32),
                       pltpu.SemaphoreType.DMA],
    )
    def k(x_hbm, pos_hbm, xs_hbm, idx_v, rows_v, sem):
        wid = lax.axis_index("s") * 2 + lax.axis_index("c")
        pltpu.sync_copy(pos_hbm.at[wid], idx_v)
        base = lax.rem(wid, 16) * _LANES
        pltpu.async_copy(x_hbm.at[pl.ds(base, _LANES)], rows_v, sem).wait()
        pltpu.sync_copy(rows_v, xs_hbm.at[idx_v])

    return k(x, pos_slot)


def _sc_combine(ys, pa, pb):
    mesh = plsc.VectorSubcoreMesh(core_axis_name="c", subcore_axis_name="s")
    otype = (jax.ShapeDtypeStruct((_B, _D), jnp.float32),
             jax.ShapeDtypeStruct((_B, _D), jnp.float32))

    @functools.partial(
        pl.kernel, mesh=mesh, out_type=otype,
        scratch_types=[pltpu.VMEM((64,), jnp.int32),
                       pltpu.VMEM((64,), jnp.int32),
                       pltpu.VMEM((64, _D), jnp.float32),
                       pltpu.VMEM((64, _D), jnp.float32),
                       pltpu.SemaphoreType.DMA,
                       pltpu.SemaphoreType.DMA],
    )
    def k(ys_hbm, pa_hbm, pb_hbm, a_hbm, b_hbm, ia_v, ib_v, ra_v, rb_v,
          sa, sb):
        wid = lax.axis_index("s") * 2 + lax.axis_index("c")
        pltpu.sync_copy(pa_hbm.at[wid], ia_v)
        pltpu.sync_copy(pb_hbm.at[wid], ib_v)
        ca = pltpu.async_copy(ys_hbm.at[ia_v], ra_v, sa)
        cb = pltpu.async_copy(ys_hbm.at[ib_v], rb_v, sb)
        ca.wait()
        pltpu.sync_copy(ra_v, a_hbm.at[pl.ds(wid * 64, 64)])
        cb.wait()
        pltpu.sync_copy(rb_v, b_hbm.at[pl.ds(wid * 64, 64)])

    return k(ys, pa, pb)


def kernel(x, Wg, bg, W1, b1, W2, b2):
    bt = _B // 8
    wg_pad = jnp.zeros((_D, _LANES), jnp.float32).at[:, :_E].set(Wg)
    bg_pad = jnp.full((1, _LANES), -1e30, jnp.float32).at[0, :_E].set(bg)

    eout, wout = pl.pallas_call(
        _gate_body,
        grid=(8,),
        in_specs=[
            pl.BlockSpec((bt, _D), lambda t: (t, 0)),
            pl.BlockSpec((_D, _LANES), lambda t: (0, 0)),
            pl.BlockSpec((1, _LANES), lambda t: (0, 0)),
        ],
        out_specs=[
            pl.BlockSpec((bt, _E), lambda t: (t, 0)),
            pl.BlockSpec((bt, _E), lambda t: (t, 0)),
        ],
        out_shape=(jax.ShapeDtypeStruct((_B, _E), jnp.int32),
                   jax.ShapeDtypeStruct((_B, _E), jnp.float32)),
    )(x, wg_pad, bg_pad)

    e_slot = jnp.concatenate([eout[:, 0], eout[:, 1]]).reshape(_SROWS, _LANES)

    pos_slot, meta = pl.pallas_call(
        _sort_body,
        out_shape=(jax.ShapeDtypeStruct((_SROWS, _LANES), jnp.int32),
                   jax.ShapeDtypeStruct((8, _LANES), jnp.int32)),
    )(e_slot)

    offs = meta[0, :_E]
    cnt = meta[1, :_E]
    end = offs + cnt
    t0 = offs // _T
    t1 = jnp.where(cnt > 0, (end - 1) // _T, t0 - 1)
    ni = jnp.maximum(t1 - t0 + 1, 0)
    base = jnp.concatenate([jnp.zeros((1,), ni.dtype), jnp.cumsum(ni)])
    gidx = jnp.arange(_G, dtype=jnp.int32)
    e_of = jnp.sum((gidx[:, None] >= base[None, 1:]).astype(jnp.int32),
                   axis=1)
    valid = gidx < base[_E]
    e_of = jnp.clip(e_of, 0, _E - 1)
    t_of = t0[e_of] + (gidx - base[e_of])
    t_of = jnp.where(valid, t_of, _NT - 1)
    rs = t_of * _T
    lo = jnp.where(valid, jnp.clip(offs[e_of] - rs, 0, _T), 0)
    hi = jnp.where(valid, jnp.clip(end[e_of] - rs, 0, _T), 0)
    wi_t = t_of.astype(jnp.int32)
    wi_e = jnp.where(valid, e_of, _E - 1).astype(jnp.int32)

    xs = _sc_dispatch(x, pos_slot)

    ys = pl.pallas_call(
        _ffn_body,
        grid_spec=pltpu.PrefetchScalarGridSpec(
            num_scalar_prefetch=4,
            grid=(_G,),
            in_specs=[
                pl.BlockSpec((_T, _D), lambda g, t, e, lo_, hi_: (t[g], 0)),
                pl.BlockSpec((1, _D, _F),
                             lambda g, t, e, lo_, hi_: (e[g], 0, 0)),
                pl.BlockSpec((1, 1, _F),
                             lambda g, t, e, lo_, hi_: (e[g], 0, 0)),
                pl.BlockSpec((1, _F, _D),
                             lambda g, t, e, lo_, hi_: (e[g], 0, 0)),
                pl.BlockSpec((1, 1, _D),
                             lambda g, t, e, lo_, hi_: (e[g], 0, 0)),
            ],
            out_specs=pl.BlockSpec((_T, _D),
                                   lambda g, t, e, lo_, hi_: (t[g], 0)),
        ),
        out_shape=jax.ShapeDtypeStruct((_S, _D), jnp.float32),
        compiler_params=pltpu.CompilerParams(
            dimension_semantics=("arbitrary",)),
    )(wi_t, wi_e, lo.astype(jnp.int32), hi.astype(jnp.int32),
      xs, W1, b1.reshape(_E, 1, _F), W2, b2.reshape(_E, 1, _D))

    pa = pos_slot[:_SROWS // 2].reshape(32, 64)
    pb = pos_slot[_SROWS // 2:].reshape(32, 64)
    a_rows, b_rows = _sc_combine(ys, pa, pb)

    out = pl.pallas_call(
        _comb_body,
        grid=(4,),
        in_specs=[
            pl.BlockSpec((_B // 4, _D), lambda t: (t, 0)),
            pl.BlockSpec((_B // 4, _D), lambda t: (t, 0)),
            pl.BlockSpec((_B // 4, 1), lambda t: (t, 0)),
            pl.BlockSpec((_B // 4, 1), lambda t: (t, 0)),
        ],
        out_specs=pl.BlockSpec((_B // 4, _D), lambda t: (t, 0)),
        out_shape=jax.ShapeDtypeStruct((_B, _D), jnp.float32),
    )(a_rows, b_rows, wout[:, 0:1], wout[:, 1:2])
    return out
